# baseline (speedup 1.0000x reference)
import functools

import jax
import jax.numpy as jnp
from jax import lax
from jax.experimental import pallas as pl
from jax.experimental.pallas import tpu as pltpu
from jax.experimental.pallas import tpu_sc as plsc

N = 10000
NPAD = 10240
E = 160000
TWOE = 2 * E
NW = 32
C = TWOE // NW
BLK = 128
ALPHA = 0.2


def _tc1_body(nf_ref, ef_ref, wn_ref, a3_ref, wep_ref, hv_ref, scal_ref):
    hv = jnp.dot(nf_ref[...], wn_ref[...], preferred_element_type=jnp.float32)
    hv_ref[...] = hv
    scal_ref[...] = (
        jnp.dot(hv, a3_ref[...], preferred_element_type=jnp.float32)
        + jnp.dot(ef_ref[...], wep_ref[...], preferred_element_type=jnp.float32))


def _tc2_body(eft_ref, wt_ref, at_ref, q2_ref):
    w = wt_ref[:, 0:16]
    a = at_ref[0:1, 0:16]
    rows = []
    for b in range(8):
        t = jnp.dot(w, eft_ref[b], preferred_element_type=jnp.float32)
        rows.append(jnp.dot(a, t, preferred_element_type=jnp.float32))
    q2_ref[...] = jnp.concatenate(rows, axis=0)


def _tc3_body(pe_ref, nn_ref, ne_ref, esum_ref, vn_ref, ve_ref):
    esum_ref[...] = pe_ref[0] + pe_ref[1]
    denom = jnp.float32(TWOE)
    x = nn_ref[...]
    m = jnp.sum(x) / denom
    vn_ref[...] = (jnp.sum((x - m) * (x - m)) / (denom - 1.0)).reshape(1, 1)
    y = ne_ref[...]
    my = jnp.sum(y) / denom
    ve_ref[...] = (jnp.sum((y - my) * (y - my)) / (denom - 1.0)).reshape(1, 1)


def _sca_body(src_hbm, dst_hbm, q2_hbm, p1_hbm, p2_hbm, q1_hbm,
              na_out, ea_out, partial_out,
              srcv, dstv, q2v, p1v, p2v, q1v, naev, eaev,
              psumv, esumv, cntv, slab):
    c = lax.axis_index("c")
    s = lax.axis_index("s")
    wid = s * 2 + c
    base = wid * C

    pltpu.sync_copy(src_hbm.at[pl.ds(base, C)], srcv)
    pltpu.sync_copy(dst_hbm.at[pl.ds(base, C)], dstv)
    pltpu.sync_copy(q2_hbm.at[pl.ds(base, C)], q2v)
    pltpu.sync_copy(p1_hbm, p1v)
    pltpu.sync_copy(p2_hbm, p2v)
    pltpu.sync_copy(q1_hbm, q1v)

    zero16 = jnp.zeros((16,), jnp.float32)

    def zbody(i, _):
        psumv[pl.ds(i * 16, 16)] = zero16
        esumv[pl.ds(i * 16, 16)] = zero16
        cntv[pl.ds(i * 16, 16)] = zero16
        return 0
    lax.fori_loop(0, NPAD // 16, zbody, 0)

    ones16 = jnp.full((16,), 1.0, jnp.float32)

    def gbody(g, _):
        off = g * 16
        s16 = srcv[pl.ds(off, 16)]
        d16 = dstv[pl.ds(off, 16)]
        q2g = q2v[pl.ds(off, 16)]
        a = plsc.load_gather(p1v, [s16])
        b = plsc.load_gather(p2v, [d16])
        cq = plsc.load_gather(q1v, [s16])
        na = a + b
        ea = cq + q2g
        na = jnp.where(na >= 0.0, na, ALPHA * na)
        ea = jnp.where(ea >= 0.0, ea, ALPHA * ea)
        na = jnp.minimum(jnp.maximum(na, -2.0), 2.0)
        ea = jnp.minimum(jnp.maximum(ea, -2.0), 2.0)
        nae = jnp.exp(na)
        eae = jnp.exp(ea)
        naev[pl.ds(off, 16)] = nae
        eaev[pl.ds(off, 16)] = eae
        plsc.addupdate_scatter(psumv, [s16], nae)
        plsc.addupdate_scatter(esumv, [s16], eae)
        plsc.addupdate_scatter(cntv, [s16], ones16)
        return 0
    lax.fori_loop(0, C // 16, gbody, 0)

    pltpu.sync_copy(naev, na_out.at[pl.ds(base, C)])
    pltpu.sync_copy(eaev, ea_out.at[pl.ds(base, C)])

    segw = NPAD // 16
    seg = s * segw
    for k, arr in enumerate((psumv, esumv, cntv)):
        pltpu.sync_copy(arr, slab.at[s])
        plsc.subcore_barrier()
        pltpu.sync_copy(slab.at[0, pl.ds(seg, segw)], naev.at[pl.ds(0, segw)])
        for t in range(1, 16):
            pltpu.sync_copy(slab.at[t, pl.ds(seg, segw)],
                            eaev.at[pl.ds(0, segw)])

            def abody(i, _):
                o = i * 16
                naev[pl.ds(o, 16)] = naev[pl.ds(o, 16)] + eaev[pl.ds(o, 16)]
                return 0
            lax.fori_loop(0, segw // 16, abody, 0)
        pltpu.sync_copy(naev.at[pl.ds(0, segw)],
                        partial_out.at[c, pl.ds(k * NPAD + seg, segw)])
        plsc.subcore_barrier()


_BITS = [8192, 4096, 2048, 1024, 512, 256, 128, 64, 32, 16, 8, 4, 2, 1]


def _scb1_body(na_hbm, ea_hbm, partial_hbm,
               normn_out, norme_out,
               naev, eaev, psumv, esumv, startsv, tmpv):
    c = lax.axis_index("c")
    s = lax.axis_index("s")
    wid = s * 2 + c
    base = wid * C

    pltpu.sync_copy(na_hbm.at[pl.ds(base, C)], naev)
    pltpu.sync_copy(ea_hbm.at[pl.ds(base, C)], eaev)

    def _combine(dref, off):
        pltpu.sync_copy(partial_hbm.at[0, pl.ds(off, NPAD)], dref)
        pltpu.sync_copy(partial_hbm.at[1, pl.ds(off, NPAD)], tmpv)

        def addb(i, _):
            o = i * 16
            dref[pl.ds(o, 16)] = dref[pl.ds(o, 16)] + tmpv[pl.ds(o, 16)]
            return 0
        lax.fori_loop(0, NPAD // 16, addb, 0)

    _combine(psumv, 0)
    _combine(esumv, NPAD)
    _combine(startsv, 2 * NPAD)

    def csbody(i, carry):
        o = i * 16
        cv = startsv[pl.ds(o, 16)]
        cs = plsc.cumsum(cv)
        startsv[pl.ds(o, 16)] = cs - cv + carry
        return carry + jnp.sum(cv)
    lax.fori_loop(0, NPAD // 16, csbody, jnp.float32(0.0))

    def nbody(g, _):
        off = g * 16
        nae = naev[pl.ds(off, 16)]
        eae = eaev[pl.ds(off, 16)]
        posf = (base + off + lax.iota(jnp.int32, 16)).astype(jnp.float32)
        lo = jnp.zeros((16,), jnp.int32)
        for bit in _BITS:
            probe = lo + bit
            pidx = jnp.minimum(probe, NPAD) - 1
            sval = plsc.load_gather(startsv, [pidx])
            ok = (probe <= NPAD) & (sval <= posf)
            lo = jnp.where(ok, probe, lo)
        r = jnp.maximum(lo - 1, 0)
        divn = plsc.load_gather(psumv, [r])
        dive = plsc.load_gather(esumv, [r])
        naev[pl.ds(off, 16)] = nae / divn
        eaev[pl.ds(off, 16)] = eae / dive
        return 0
    lax.fori_loop(0, C // 16, nbody, 0)

    pltpu.sync_copy(naev, normn_out.at[pl.ds(base, C)])
    pltpu.sync_copy(eaev, norme_out.at[pl.ds(base, C)])


NOWN = NPAD // NW
CB = 8000
NCH = TWOE // CB


def _scb2_body(src_hbm, dst_hbm, nn_hbm, hv_hbm,
               nout_hbm,
               srcv, dstv, wv, dstm, um, wm, rowsv, accf, semg, semc):
    c = lax.axis_index("c")
    s = lax.axis_index("s")
    wid = s * 2 + c
    lo = wid * NOWN
    hi = lo + NOWN

    zero16 = jnp.zeros((16,), jnp.float32)
    zero16i = jnp.zeros((16,), jnp.int32)
    iota16 = lax.iota(jnp.int32, 16)

    def z1(i, _):
        o = i * 16
        dstm[pl.ds(o, 16)] = zero16i
        um[pl.ds(o, 16)] = zero16i
        wm[pl.ds(o, 16)] = zero16
        return 0
    lax.fori_loop(0, (CB + 16) // 16, z1, 0)

    def z2(i, _):
        accf[pl.ds(i * 16, 16)] = zero16
        return 0
    lax.fori_loop(0, (NOWN * 128) // 16, z2, 0)

    def chunk_body(ch, _):
        pltpu.async_copy(src_hbm.at[pl.ds(ch * CB, CB)], srcv, semc)
        pltpu.async_copy(dst_hbm.at[pl.ds(ch * CB, CB)], dstv, semc)
        pltpu.async_copy(nn_hbm.at[pl.ds(ch * CB, CB)], wv, semc)
        pltpu.make_async_copy(src_hbm.at[pl.ds(ch * CB, CB)], srcv, semc).wait()
        pltpu.make_async_copy(dst_hbm.at[pl.ds(ch * CB, CB)], dstv, semc).wait()
        pltpu.make_async_copy(nn_hbm.at[pl.ds(ch * CB, CB)], wv, semc).wait()

        def scan(g, ptrv):
            off = g * 16
            s16 = srcv[pl.ds(off, 16)]
            m = (s16 >= lo) & (s16 < hi)
            mi = jnp.where(m, 1, 0).astype(jnp.int32)
            pos = ptrv + plsc.cumsum(mi) - mi
            plsc.store_scatter(dstm, [pos], dstv[pl.ds(off, 16)], mask=m)
            plsc.store_scatter(um, [pos], s16 - lo, mask=m)
            plsc.store_scatter(wm, [pos], wv[pl.ds(off, 16)], mask=m)
            return ptrv + plsc.all_reduce_population_count(m)
        ptrv = lax.fori_loop(0, CB // 16, scan,
                             jnp.zeros((16,), jnp.int32))
        ptr = jnp.max(ptrv)

        nb = (ptr + BLK - 1) // BLK

        def process(b, par):
            rows_here = jnp.minimum(BLK, ptr - b * BLK)

            def rbody(j, _):
                e = b * BLK + j
                spl = jnp.full((16,), e, jnp.int32)
                u16 = plsc.load_gather(um, [spl])
                w16 = plsc.load_gather(wm, [spl])
                base16 = u16 * 128 + iota16
                for sg in range(8):
                    o = sg * 16
                    val = rowsv[par, j, pl.ds(o, 16)] * w16
                    plsc.addupdate_scatter(accf, [base16 + o], val)
                return 0
            lax.fori_loop(0, rows_here, rbody, 0)

        def pair(q, _):
            b0 = 2 * q
            b1 = b0 + 1
            pltpu.async_copy(hv_hbm.at[dstm.at[pl.ds(b0 * BLK, BLK)]],
                             rowsv.at[0], semg)

            @pl.when(b1 < nb)
            def _fire1():
                pltpu.async_copy(hv_hbm.at[dstm.at[pl.ds(b1 * BLK, BLK)]],
                                 rowsv.at[1], semg)
            pltpu.make_async_copy(hv_hbm.at[dstm.at[pl.ds(b0 * BLK, BLK)]],
                                  rowsv.at[0], semg).wait()
            process(b0, 0)

            @pl.when(b1 < nb)
            def _do1():
                pltpu.make_async_copy(
                    hv_hbm.at[dstm.at[pl.ds(b1 * BLK, BLK)]],
                    rowsv.at[1], semg).wait()
                process(b1, 1)
            return 0
        lax.fori_loop(0, (nb + 1) // 2, pair, 0)
        return 0
    lax.fori_loop(0, NCH, chunk_body, 0)

    pltpu.sync_copy(accf, nout_hbm.at[pl.ds(lo * 128, NOWN * 128)])


def _scc_body(src_hbm, dst_hbm, ne_hbm, evt_hbm, partial_out,
              srcv, dstv, nev, colv, colv2, accv, accv2, slab):
    c = lax.axis_index("c")
    s = lax.axis_index("s")
    wid = s * 2 + c
    base = wid * C

    pltpu.sync_copy(src_hbm.at[pl.ds(base, C)], srcv)
    pltpu.sync_copy(dst_hbm.at[pl.ds(base, C)], dstv)
    pltpu.sync_copy(ne_hbm.at[pl.ds(base, C)], nev)

    zero16 = jnp.zeros((16,), jnp.float32)
    segw = NPAD // 16
    seg = s * segw

    segw2 = 2 * segw
    seg2 = s * segw2
    for k in range(8):
        pltpu.sync_copy(evt_hbm.at[2 * k], colv)
        pltpu.sync_copy(evt_hbm.at[2 * k + 1], colv2)

        def zb(i, _):
            o = i * 16
            accv[pl.ds(o, 16)] = zero16
            accv2[pl.ds(o, 16)] = zero16
            return 0
        lax.fori_loop(0, NPAD // 16, zb, 0)

        def gb(g, _):
            off = g * 16
            d16 = dstv[pl.ds(off, 16)]
            s16 = srcv[pl.ds(off, 16)]
            w = nev[pl.ds(off, 16)]
            v = plsc.load_gather(colv, [d16]) * w
            plsc.addupdate_scatter(accv, [s16], v)
            v2 = plsc.load_gather(colv2, [d16]) * w
            plsc.addupdate_scatter(accv2, [s16], v2)
            return 0
        lax.fori_loop(0, C // 16, gb, 0)

        pltpu.sync_copy(accv, slab.at[s, pl.ds(0, NPAD)])
        pltpu.sync_copy(accv2, slab.at[s, pl.ds(NPAD, NPAD)])
        plsc.subcore_barrier()
        pltpu.sync_copy(slab.at[0, pl.ds(seg2, segw2)],
                        colv.at[pl.ds(0, segw2)])
        for t in range(1, 16):
            pltpu.sync_copy(slab.at[t, pl.ds(seg2, segw2)],
                            accv.at[pl.ds(0, segw2)])

            def ab(i, _):
                o = i * 16
                colv[pl.ds(o, 16)] = colv[pl.ds(o, 16)] + accv[pl.ds(o, 16)]
                return 0
            lax.fori_loop(0, segw2 // 16, ab, 0)
        pltpu.sync_copy(colv.at[pl.ds(0, segw2)],
                        partial_out.at[c, pl.ds(k * 2 * NPAD + seg2, segw2)])
        plsc.subcore_barrier()


@jax.jit
def kernel(node_fts, edge_fts, edges, W_node, W_edge, a_node, a_edge):
    node_fts = jnp.squeeze(node_fts)
    edge_fts = jnp.squeeze(edge_fts)
    edges = jnp.squeeze(edges)

    f32 = jnp.float32
    edges2 = edges.reshape(E, 2)
    src_d = jnp.concatenate([edges2[:, 0], edges2[:, 1]])
    dst_d = jnp.concatenate([edges2[:, 1], edges2[:, 0]])

    a3 = jnp.zeros((128, 128), f32)
    a3 = a3.at[:, 0].set(a_node[:128, 0])
    a3 = a3.at[:, 1].set(a_node[128:, 0])
    a3 = a3.at[:, 2].set(a_edge[:128, 0])
    wep = jnp.zeros((16, 128), f32).at[:, 16:32].set(W_edge)
    wT = jnp.zeros((16, 128), f32).at[:, 0:16].set(W_edge.T)
    a2eT = jnp.zeros((8, 128), f32).at[0:1, 0:16].set(a_edge[128:].T)

    hv, scal = pl.pallas_call(
        _tc1_body,
        grid=(5,),
        in_specs=[
            pl.BlockSpec((2000, 128), lambda i: (i, 0)),
            pl.BlockSpec((2000, 16), lambda i: (i, 0)),
            pl.BlockSpec((128, 128), lambda i: (0, 0)),
            pl.BlockSpec((128, 128), lambda i: (0, 0)),
            pl.BlockSpec((16, 128), lambda i: (0, 0)),
        ],
        out_specs=[
            pl.BlockSpec((2000, 128), lambda i: (i, 0)),
            pl.BlockSpec((2000, 128), lambda i: (i, 0)),
        ],
        out_shape=[
            jax.ShapeDtypeStruct((N, 128), f32),
            jax.ShapeDtypeStruct((N, 128), f32),
        ],
    )(node_fts, edge_fts[:N], W_node, a3, wep)

    eft = edge_fts.reshape(1250, 128, 16).transpose(0, 2, 1)
    eft = jnp.pad(eft, ((0, 30), (0, 0), (0, 0)))
    q2_2d = pl.pallas_call(
        _tc2_body,
        grid=(160,),
        in_specs=[
            pl.BlockSpec((8, 16, 128), lambda i: (i, 0, 0)),
            pl.BlockSpec((16, 128), lambda i: (0, 0)),
            pl.BlockSpec((8, 128), lambda i: (0, 0)),
        ],
        out_specs=pl.BlockSpec((8, 128), lambda i: (i, 0)),
        out_shape=jax.ShapeDtypeStruct((1280, 128), f32),
    )(eft, wT, a2eT)
    q2 = q2_2d.reshape(1280 * 128)[:E]
    q2_d = jnp.concatenate([q2, q2])

    p1 = jnp.pad(scal[:, 0], (0, NPAD - N))
    p2 = jnp.pad(scal[:, 1], (0, NPAD - N))
    q1 = jnp.pad(scal[:, 2], (0, NPAD - N))
    evT = jnp.pad(scal[:, 16:32], ((0, NPAD - N), (0, 0))).T

    src_d, dst_d, q2_d, p1, p2, q1, evT, hv = (
        lax.optimization_barrier(
            (src_d, dst_d, q2_d, p1, p2, q1, evT, hv)))

    mesh = plsc.VectorSubcoreMesh(core_axis_name="c", subcore_axis_name="s")
    sc_params = pltpu.CompilerParams(needs_layout_passes=False)

    sca = functools.partial(
        pl.kernel, _sca_body, mesh=mesh,
        compiler_params=sc_params,
        out_type=[
            jax.ShapeDtypeStruct((TWOE,), f32),
            jax.ShapeDtypeStruct((TWOE,), f32),
            jax.ShapeDtypeStruct((2, 3 * NPAD), f32),
        ],
        scratch_types=[
            pltpu.VMEM((C,), jnp.int32),
            pltpu.VMEM((C,), jnp.int32),
            pltpu.VMEM((C,), f32),
            pltpu.VMEM((NPAD,), f32),
            pltpu.VMEM((NPAD,), f32),
            pltpu.VMEM((NPAD,), f32),
            pltpu.VMEM((C,), f32),
            pltpu.VMEM((C,), f32),
            pltpu.VMEM((NPAD,), f32),
            pltpu.VMEM((NPAD,), f32),
            pltpu.VMEM((NPAD,), f32),
            pltpu.VMEM_SHARED((16, NPAD), f32),
        ],
    )()
    na_e, ea_e, partials = sca(src_d, dst_d, q2_d, p1, p2, q1)
    na_e, ea_e, partials = lax.optimization_barrier((na_e, ea_e, partials))

    scb1 = functools.partial(
        pl.kernel, _scb1_body, mesh=mesh,
        compiler_params=sc_params,
        out_type=[
            jax.ShapeDtypeStruct((TWOE,), f32),
            jax.ShapeDtypeStruct((TWOE,), f32),
        ],
        scratch_types=[
            pltpu.VMEM((C,), f32),
            pltpu.VMEM((C,), f32),
            pltpu.VMEM((NPAD,), f32),
            pltpu.VMEM((NPAD,), f32),
            pltpu.VMEM((NPAD,), f32),
            pltpu.VMEM((NPAD,), f32),
        ],
    )()
    normn, norme = scb1(na_e, ea_e, partials)
    normn, norme = lax.optimization_barrier((normn, norme))

    scb2 = functools.partial(
        pl.kernel, _scb2_body, mesh=mesh,
        compiler_params=sc_params,
        out_type=jax.ShapeDtypeStruct((NPAD * 128,), f32),
        scratch_types=[
            pltpu.VMEM((CB,), jnp.int32),
            pltpu.VMEM((CB,), jnp.int32),
            pltpu.VMEM((CB,), f32),
            pltpu.VMEM((CB + 16,), jnp.int32),
            pltpu.VMEM((CB + 16,), jnp.int32),
            pltpu.VMEM((CB + 16,), f32),
            pltpu.VMEM((2, BLK, 128), f32),
            pltpu.VMEM((NOWN * 128,), f32),
            pltpu.SemaphoreType.DMA,
            pltpu.SemaphoreType.DMA,
        ],
    )()
    nacc = scb2(src_d, dst_d, normn, hv)
    norme_b = norme

    scc = functools.partial(
        pl.kernel, _scc_body, mesh=mesh,
        compiler_params=sc_params,
        out_type=jax.ShapeDtypeStruct((2, 16 * NPAD), f32),
        scratch_types=[
            pltpu.VMEM((C,), jnp.int32),
            pltpu.VMEM((C,), jnp.int32),
            pltpu.VMEM((C,), f32),
            pltpu.VMEM((NPAD,), f32),
            pltpu.VMEM((NPAD,), f32),
            pltpu.VMEM((NPAD,), f32),
            pltpu.VMEM((NPAD,), f32),
            pltpu.VMEM_SHARED((16, 2 * NPAD), f32),
        ],
    )()
    pe = scc(src_d, dst_d, norme_b, evT).reshape(2, 16, NPAD)

    nn2 = normn.reshape(2500, 128)
    ne2 = norme.reshape(2500, 128)
    esum, varn, vare = pl.pallas_call(
        _tc3_body,
        grid=(1,),
        in_specs=[
            pl.BlockSpec((2, 16, NPAD), lambda i: (0, 0, 0)),
            pl.BlockSpec((2500, 128), lambda i: (0, 0)),
            pl.BlockSpec((2500, 128), lambda i: (0, 0)),
        ],
        out_specs=[
            pl.BlockSpec((16, NPAD), lambda i: (0, 0)),
            pl.BlockSpec((1, 1), lambda i: (0, 0)),
            pl.BlockSpec((1, 1), lambda i: (0, 0)),
        ],
        out_shape=[
            jax.ShapeDtypeStruct((16, NPAD), f32),
            jax.ShapeDtypeStruct((1, 1), f32),
            jax.ShapeDtypeStruct((1, 1), f32),
        ],
    )(pe, nn2, ne2)

    node_out = nacc.reshape(NPAD, 128)[:N]
    edge_out = esum.T[:N]
    return node_out, edge_out, varn[0, 0], vare[0, 0]


# speedup vs baseline: 1.4250x; 1.4250x over previous
import functools

import jax
import jax.numpy as jnp
from jax import lax
from jax.experimental import pallas as pl
from jax.experimental.pallas import tpu as pltpu
from jax.experimental.pallas import tpu_sc as plsc

N = 10000
NPAD = 10240
E = 160000
TWOE = 2 * E
NW = 32
C = TWOE // NW
BLK = 128
ALPHA = 0.2


def _tc1_body(nf_ref, ef_ref, wn_ref, a3_ref, wep_ref, hv_ref, scal_ref):
    hv = jnp.dot(nf_ref[...], wn_ref[...], preferred_element_type=jnp.float32)
    hv_ref[...] = hv
    scal_ref[...] = (
        jnp.dot(hv, a3_ref[...], preferred_element_type=jnp.float32)
        + jnp.dot(ef_ref[...], wep_ref[...], preferred_element_type=jnp.float32))


def _tc2_body(eft_ref, wt_ref, at_ref, q2_ref):
    w = wt_ref[:, 0:16]
    a = at_ref[0:1, 0:16]
    rows = []
    for b in range(8):
        t = jnp.dot(w, eft_ref[b], preferred_element_type=jnp.float32)
        rows.append(jnp.dot(a, t, preferred_element_type=jnp.float32))
    q2_ref[...] = jnp.concatenate(rows, axis=0)


def _tc3_body(pe_ref, nn_ref, ne_ref, esum_ref, vn_ref, ve_ref):
    esum_ref[...] = pe_ref[0] + pe_ref[1]
    denom = jnp.float32(TWOE)
    x = nn_ref[...]
    m = jnp.sum(x) / denom
    vn_ref[...] = (jnp.sum((x - m) * (x - m)) / (denom - 1.0)).reshape(1, 1)
    y = ne_ref[...]
    my = jnp.sum(y) / denom
    ve_ref[...] = (jnp.sum((y - my) * (y - my)) / (denom - 1.0)).reshape(1, 1)


def _sca_body(src_hbm, dst_hbm, q2_hbm, p1_hbm, p2_hbm, q1_hbm,
              na_out, ea_out, partial_out,
              srcv, dstv, q2v, p1v, p2v, q1v, naev, eaev,
              psumv, esumv, cntv, slab):
    c = lax.axis_index("c")
    s = lax.axis_index("s")
    wid = s * 2 + c
    base = wid * C

    pltpu.sync_copy(src_hbm.at[pl.ds(base, C)], srcv)
    pltpu.sync_copy(dst_hbm.at[pl.ds(base, C)], dstv)
    pltpu.sync_copy(q2_hbm.at[pl.ds(base, C)], q2v)
    pltpu.sync_copy(p1_hbm, p1v)
    pltpu.sync_copy(p2_hbm, p2v)
    pltpu.sync_copy(q1_hbm, q1v)

    zero16 = jnp.zeros((16,), jnp.float32)

    def zbody(i, _):
        psumv[pl.ds(i * 16, 16)] = zero16
        esumv[pl.ds(i * 16, 16)] = zero16
        cntv[pl.ds(i * 16, 16)] = zero16
        return 0
    lax.fori_loop(0, NPAD // 16, zbody, 0)

    ones16 = jnp.full((16,), 1.0, jnp.float32)

    def gbody(g, _):
        off = g * 16
        s16 = srcv[pl.ds(off, 16)]
        d16 = dstv[pl.ds(off, 16)]
        q2g = q2v[pl.ds(off, 16)]
        a = plsc.load_gather(p1v, [s16])
        b = plsc.load_gather(p2v, [d16])
        cq = plsc.load_gather(q1v, [s16])
        na = a + b
        ea = cq + q2g
        na = jnp.where(na >= 0.0, na, ALPHA * na)
        ea = jnp.where(ea >= 0.0, ea, ALPHA * ea)
        na = jnp.minimum(jnp.maximum(na, -2.0), 2.0)
        ea = jnp.minimum(jnp.maximum(ea, -2.0), 2.0)
        nae = jnp.exp(na)
        eae = jnp.exp(ea)
        naev[pl.ds(off, 16)] = nae
        eaev[pl.ds(off, 16)] = eae
        plsc.addupdate_scatter(psumv, [s16], nae)
        plsc.addupdate_scatter(esumv, [s16], eae)
        plsc.addupdate_scatter(cntv, [s16], ones16)
        return 0
    lax.fori_loop(0, C // 16, gbody, 0)

    pltpu.sync_copy(naev, na_out.at[pl.ds(base, C)])
    pltpu.sync_copy(eaev, ea_out.at[pl.ds(base, C)])

    segw = NPAD // 16
    seg = s * segw
    for k, arr in enumerate((psumv, esumv, cntv)):
        pltpu.sync_copy(arr, slab.at[s])
        plsc.subcore_barrier()
        pltpu.sync_copy(slab.at[0, pl.ds(seg, segw)], naev.at[pl.ds(0, segw)])
        for t in range(1, 16):
            pltpu.sync_copy(slab.at[t, pl.ds(seg, segw)],
                            eaev.at[pl.ds(0, segw)])

            def abody(i, _):
                o = i * 16
                naev[pl.ds(o, 16)] = naev[pl.ds(o, 16)] + eaev[pl.ds(o, 16)]
                return 0
            lax.fori_loop(0, segw // 16, abody, 0)
        pltpu.sync_copy(naev.at[pl.ds(0, segw)],
                        partial_out.at[c, pl.ds(k * NPAD + seg, segw)])
        plsc.subcore_barrier()


_BITS = [8192, 4096, 2048, 1024, 512, 256, 128, 64, 32, 16, 8, 4, 2, 1]


def _scb1_body(na_hbm, ea_hbm, partial_hbm,
               normn_out, norme_out,
               naev, eaev, psumv, esumv, startsv, tmpv):
    c = lax.axis_index("c")
    s = lax.axis_index("s")
    wid = s * 2 + c
    base = wid * C

    pltpu.sync_copy(na_hbm.at[pl.ds(base, C)], naev)
    pltpu.sync_copy(ea_hbm.at[pl.ds(base, C)], eaev)

    def _combine(dref, off):
        pltpu.sync_copy(partial_hbm.at[0, pl.ds(off, NPAD)], dref)
        pltpu.sync_copy(partial_hbm.at[1, pl.ds(off, NPAD)], tmpv)

        def addb(i, _):
            o = i * 16
            dref[pl.ds(o, 16)] = dref[pl.ds(o, 16)] + tmpv[pl.ds(o, 16)]
            return 0
        lax.fori_loop(0, NPAD // 16, addb, 0)

    _combine(psumv, 0)
    _combine(esumv, NPAD)
    _combine(startsv, 2 * NPAD)

    def csbody(i, carry):
        o = i * 16
        cv = startsv[pl.ds(o, 16)]
        cs = plsc.cumsum(cv)
        startsv[pl.ds(o, 16)] = cs - cv + carry
        return carry + jnp.sum(cv)
    lax.fori_loop(0, NPAD // 16, csbody, jnp.float32(0.0))

    def nbody(g, _):
        off = g * 16
        nae = naev[pl.ds(off, 16)]
        eae = eaev[pl.ds(off, 16)]
        posf = (base + off + lax.iota(jnp.int32, 16)).astype(jnp.float32)
        lo = jnp.zeros((16,), jnp.int32)
        for bit in _BITS:
            probe = lo + bit
            pidx = jnp.minimum(probe, NPAD) - 1
            sval = plsc.load_gather(startsv, [pidx])
            ok = (probe <= NPAD) & (sval <= posf)
            lo = jnp.where(ok, probe, lo)
        r = jnp.maximum(lo - 1, 0)
        divn = plsc.load_gather(psumv, [r])
        dive = plsc.load_gather(esumv, [r])
        naev[pl.ds(off, 16)] = nae / divn
        eaev[pl.ds(off, 16)] = eae / dive
        return 0
    lax.fori_loop(0, C // 16, nbody, 0)

    pltpu.sync_copy(naev, normn_out.at[pl.ds(base, C)])
    pltpu.sync_copy(eaev, norme_out.at[pl.ds(base, C)])


NOWN = NPAD // NW
CB = 10000
NCH = TWOE // CB


def _scb2_body(src_hbm, dst_hbm, nn_hbm, hv_hbm,
               nout_hbm,
               srcv, dstv, wv, dstm, um, wm, rowsv, accf, semg, semc):
    c = lax.axis_index("c")
    s = lax.axis_index("s")
    wid = s * 2 + c
    lo = wid * NOWN
    hi = lo + NOWN

    zero16 = jnp.zeros((16,), jnp.float32)
    zero16i = jnp.zeros((16,), jnp.int32)
    iota16 = lax.iota(jnp.int32, 16)

    def z1(i, _):
        o = i * 16
        dstm[pl.ds(o, 16)] = zero16i
        um[pl.ds(o, 16)] = zero16i
        wm[pl.ds(o, 16)] = zero16
        return 0
    lax.fori_loop(0, (CB + 16) // 16, z1, 0)

    def z2(i, _):
        accf[pl.ds(i * 16, 16)] = zero16
        return 0
    lax.fori_loop(0, (NOWN * 128) // 16, z2, 0)

    def chunk_body(ch, _):
        pltpu.async_copy(src_hbm.at[pl.ds(ch * CB, CB)], srcv, semc)
        pltpu.async_copy(dst_hbm.at[pl.ds(ch * CB, CB)], dstv, semc)
        pltpu.async_copy(nn_hbm.at[pl.ds(ch * CB, CB)], wv, semc)
        pltpu.make_async_copy(src_hbm.at[pl.ds(ch * CB, CB)], srcv, semc).wait()
        pltpu.make_async_copy(dst_hbm.at[pl.ds(ch * CB, CB)], dstv, semc).wait()
        pltpu.make_async_copy(nn_hbm.at[pl.ds(ch * CB, CB)], wv, semc).wait()

        def scan(g, ptrv):
            off = g * 16
            s16 = srcv[pl.ds(off, 16)]
            m = (s16 >= lo) & (s16 < hi)
            mi = jnp.where(m, 1, 0).astype(jnp.int32)
            pos = ptrv + plsc.cumsum(mi) - mi
            plsc.store_scatter(dstm, [pos], dstv[pl.ds(off, 16)], mask=m)
            plsc.store_scatter(um, [pos], s16 - lo, mask=m)
            plsc.store_scatter(wm, [pos], wv[pl.ds(off, 16)], mask=m)
            return ptrv + plsc.all_reduce_population_count(m)
        ptrv = lax.fori_loop(0, CB // 16, scan,
                             jnp.zeros((16,), jnp.int32))
        ptr = jnp.max(ptrv)

        nb = (ptr + BLK - 1) // BLK

        def blk(b, _):
            pltpu.async_copy(hv_hbm.at[dstm.at[pl.ds(b * BLK, BLK)]],
                             rowsv, semg).wait()
            rows_here = jnp.minimum(BLK, ptr - b * BLK)

            def rbody(j, _):
                e = b * BLK + j
                spl = jnp.full((16,), e, jnp.int32)
                u16 = plsc.load_gather(um, [spl])
                w16 = plsc.load_gather(wm, [spl])
                base16 = u16 * 128 + iota16
                for sg in range(8):
                    o = sg * 16
                    val = rowsv[j, pl.ds(o, 16)] * w16
                    plsc.addupdate_scatter(accf, [base16 + o], val)
                return 0
            lax.fori_loop(0, rows_here, rbody, 0)
            return 0
        lax.fori_loop(0, nb, blk, 0)
        return 0
    lax.fori_loop(0, NCH, chunk_body, 0)

    pltpu.sync_copy(accf, nout_hbm.at[pl.ds(lo * 128, NOWN * 128)])


def _scc_body(src_hbm, dst_hbm, ne_hbm, evt_hbm, partial_out,
              srcv, dstv, nev, colv, colv2, accv, accv2, slab):
    c = lax.axis_index("c")
    s = lax.axis_index("s")
    wid = s * 2 + c
    base = wid * C

    pltpu.sync_copy(src_hbm.at[pl.ds(base, C)], srcv)
    pltpu.sync_copy(dst_hbm.at[pl.ds(base, C)], dstv)
    pltpu.sync_copy(ne_hbm.at[pl.ds(base, C)], nev)

    zero16 = jnp.zeros((16,), jnp.float32)
    segw = NPAD // 16
    seg = s * segw

    segw2 = 2 * segw
    seg2 = s * segw2
    for k in range(8):
        pltpu.sync_copy(evt_hbm.at[2 * k], colv)
        pltpu.sync_copy(evt_hbm.at[2 * k + 1], colv2)

        def zb(i, _):
            o = i * 16
            accv[pl.ds(o, 16)] = zero16
            accv2[pl.ds(o, 16)] = zero16
            return 0
        lax.fori_loop(0, NPAD // 16, zb, 0)

        def gb(g, _):
            off = g * 16
            d16 = dstv[pl.ds(off, 16)]
            s16 = srcv[pl.ds(off, 16)]
            w = nev[pl.ds(off, 16)]
            v = plsc.load_gather(colv, [d16]) * w
            plsc.addupdate_scatter(accv, [s16], v)
            v2 = plsc.load_gather(colv2, [d16]) * w
            plsc.addupdate_scatter(accv2, [s16], v2)
            return 0
        lax.fori_loop(0, C // 16, gb, 0)

        pltpu.sync_copy(accv, slab.at[s, pl.ds(0, NPAD)])
        pltpu.sync_copy(accv2, slab.at[s, pl.ds(NPAD, NPAD)])
        plsc.subcore_barrier()
        pltpu.sync_copy(slab.at[0, pl.ds(seg2, segw2)],
                        colv.at[pl.ds(0, segw2)])
        for t in range(1, 16):
            pltpu.sync_copy(slab.at[t, pl.ds(seg2, segw2)],
                            accv.at[pl.ds(0, segw2)])

            def ab(i, _):
                o = i * 16
                colv[pl.ds(o, 16)] = colv[pl.ds(o, 16)] + accv[pl.ds(o, 16)]
                return 0
            lax.fori_loop(0, segw2 // 16, ab, 0)
        pltpu.sync_copy(colv.at[pl.ds(0, segw2)],
                        partial_out.at[c, pl.ds(k * 2 * NPAD + seg2, segw2)])
        plsc.subcore_barrier()


@jax.jit
def kernel(node_fts, edge_fts, edges, W_node, W_edge, a_node, a_edge):
    node_fts = jnp.squeeze(node_fts)
    edge_fts = jnp.squeeze(edge_fts)
    edges = jnp.squeeze(edges)

    f32 = jnp.float32
    edges2 = edges.reshape(E, 2)
    src_d = jnp.concatenate([edges2[:, 0], edges2[:, 1]])
    dst_d = jnp.concatenate([edges2[:, 1], edges2[:, 0]])

    a3 = jnp.zeros((128, 128), f32)
    a3 = a3.at[:, 0].set(a_node[:128, 0])
    a3 = a3.at[:, 1].set(a_node[128:, 0])
    a3 = a3.at[:, 2].set(a_edge[:128, 0])
    wep = jnp.zeros((16, 128), f32).at[:, 16:32].set(W_edge)
    wT = jnp.zeros((16, 128), f32).at[:, 0:16].set(W_edge.T)
    a2eT = jnp.zeros((8, 128), f32).at[0:1, 0:16].set(a_edge[128:].T)

    hv, scal = pl.pallas_call(
        _tc1_body,
        grid=(5,),
        in_specs=[
            pl.BlockSpec((2000, 128), lambda i: (i, 0)),
            pl.BlockSpec((2000, 16), lambda i: (i, 0)),
            pl.BlockSpec((128, 128), lambda i: (0, 0)),
            pl.BlockSpec((128, 128), lambda i: (0, 0)),
            pl.BlockSpec((16, 128), lambda i: (0, 0)),
        ],
        out_specs=[
            pl.BlockSpec((2000, 128), lambda i: (i, 0)),
            pl.BlockSpec((2000, 128), lambda i: (i, 0)),
        ],
        out_shape=[
            jax.ShapeDtypeStruct((N, 128), f32),
            jax.ShapeDtypeStruct((N, 128), f32),
        ],
    )(node_fts, edge_fts[:N], W_node, a3, wep)

    eft = edge_fts.reshape(1250, 128, 16).transpose(0, 2, 1)
    eft = jnp.pad(eft, ((0, 30), (0, 0), (0, 0)))
    q2_2d = pl.pallas_call(
        _tc2_body,
        grid=(160,),
        in_specs=[
            pl.BlockSpec((8, 16, 128), lambda i: (i, 0, 0)),
            pl.BlockSpec((16, 128), lambda i: (0, 0)),
            pl.BlockSpec((8, 128), lambda i: (0, 0)),
        ],
        out_specs=pl.BlockSpec((8, 128), lambda i: (i, 0)),
        out_shape=jax.ShapeDtypeStruct((1280, 128), f32),
    )(eft, wT, a2eT)
    q2 = q2_2d.reshape(1280 * 128)[:E]
    q2_d = jnp.concatenate([q2, q2])

    p1 = jnp.pad(scal[:, 0], (0, NPAD - N))
    p2 = jnp.pad(scal[:, 1], (0, NPAD - N))
    q1 = jnp.pad(scal[:, 2], (0, NPAD - N))
    evT = jnp.pad(scal[:, 16:32], ((0, NPAD - N), (0, 0))).T

    src_d, dst_d, q2_d, p1, p2, q1, evT, hv = (
        lax.optimization_barrier(
            (src_d, dst_d, q2_d, p1, p2, q1, evT, hv)))

    mesh = plsc.VectorSubcoreMesh(core_axis_name="c", subcore_axis_name="s")
    sc_params = pltpu.CompilerParams(needs_layout_passes=False)

    sca = functools.partial(
        pl.kernel, _sca_body, mesh=mesh,
        compiler_params=sc_params,
        out_type=[
            jax.ShapeDtypeStruct((TWOE,), f32),
            jax.ShapeDtypeStruct((TWOE,), f32),
            jax.ShapeDtypeStruct((2, 3 * NPAD), f32),
        ],
        scratch_types=[
            pltpu.VMEM((C,), jnp.int32),
            pltpu.VMEM((C,), jnp.int32),
            pltpu.VMEM((C,), f32),
            pltpu.VMEM((NPAD,), f32),
            pltpu.VMEM((NPAD,), f32),
            pltpu.VMEM((NPAD,), f32),
            pltpu.VMEM((C,), f32),
            pltpu.VMEM((C,), f32),
            pltpu.VMEM((NPAD,), f32),
            pltpu.VMEM((NPAD,), f32),
            pltpu.VMEM((NPAD,), f32),
            pltpu.VMEM_SHARED((16, NPAD), f32),
        ],
    )()
    na_e, ea_e, partials = sca(src_d, dst_d, q2_d, p1, p2, q1)
    na_e, ea_e, partials = lax.optimization_barrier((na_e, ea_e, partials))

    scb1 = functools.partial(
        pl.kernel, _scb1_body, mesh=mesh,
        compiler_params=sc_params,
        out_type=[
            jax.ShapeDtypeStruct((TWOE,), f32),
            jax.ShapeDtypeStruct((TWOE,), f32),
        ],
        scratch_types=[
            pltpu.VMEM((C,), f32),
            pltpu.VMEM((C,), f32),
            pltpu.VMEM((NPAD,), f32),
            pltpu.VMEM((NPAD,), f32),
            pltpu.VMEM((NPAD,), f32),
            pltpu.VMEM((NPAD,), f32),
        ],
    )()
    normn, norme = scb1(na_e, ea_e, partials)
    normn, norme = lax.optimization_barrier((normn, norme))

    scb2 = functools.partial(
        pl.kernel, _scb2_body, mesh=mesh,
        compiler_params=sc_params,
        out_type=jax.ShapeDtypeStruct((NPAD * 128,), f32),
        scratch_types=[
            pltpu.VMEM((CB,), jnp.int32),
            pltpu.VMEM((CB,), jnp.int32),
            pltpu.VMEM((CB,), f32),
            pltpu.VMEM((CB + 16,), jnp.int32),
            pltpu.VMEM((CB + 16,), jnp.int32),
            pltpu.VMEM((CB + 16,), f32),
            pltpu.VMEM((BLK, 128), f32),
            pltpu.VMEM((NOWN * 128,), f32),
            pltpu.SemaphoreType.DMA,
            pltpu.SemaphoreType.DMA,
        ],
    )()
    nacc = scb2(src_d, dst_d, normn, hv)
    norme_b = norme

    scc = functools.partial(
        pl.kernel, _scc_body, mesh=mesh,
        compiler_params=sc_params,
        out_type=jax.ShapeDtypeStruct((2, 16 * NPAD), f32),
        scratch_types=[
            pltpu.VMEM((C,), jnp.int32),
            pltpu.VMEM((C,), jnp.int32),
            pltpu.VMEM((C,), f32),
            pltpu.VMEM((NPAD,), f32),
            pltpu.VMEM((NPAD,), f32),
            pltpu.VMEM((NPAD,), f32),
            pltpu.VMEM((NPAD,), f32),
            pltpu.VMEM_SHARED((16, 2 * NPAD), f32),
        ],
    )()
    pe = scc(src_d, dst_d, norme_b, evT).reshape(2, 16, NPAD)

    nn2 = normn.reshape(2500, 128)
    ne2 = norme.reshape(2500, 128)
    esum, varn, vare = pl.pallas_call(
        _tc3_body,
        grid=(1,),
        in_specs=[
            pl.BlockSpec((2, 16, NPAD), lambda i: (0, 0, 0)),
            pl.BlockSpec((2500, 128), lambda i: (0, 0)),
            pl.BlockSpec((2500, 128), lambda i: (0, 0)),
        ],
        out_specs=[
            pl.BlockSpec((16, NPAD), lambda i: (0, 0)),
            pl.BlockSpec((1, 1), lambda i: (0, 0)),
            pl.BlockSpec((1, 1), lambda i: (0, 0)),
        ],
        out_shape=[
            jax.ShapeDtypeStruct((16, NPAD), f32),
            jax.ShapeDtypeStruct((1, 1), f32),
            jax.ShapeDtypeStruct((1, 1), f32),
        ],
    )(pe, nn2, ne2)

    node_out = nacc.reshape(NPAD, 128)[:N]
    edge_out = esum.T[:N]
    return node_out, edge_out, varn[0, 0], vare[0, 0]


# speedup vs baseline: 1.4277x; 1.0018x over previous
import functools

import jax
import jax.numpy as jnp
from jax import lax
from jax.experimental import pallas as pl
from jax.experimental.pallas import tpu as pltpu
from jax.experimental.pallas import tpu_sc as plsc

N = 10000
NPAD = 10240
E = 160000
TWOE = 2 * E
NW = 32
C = TWOE // NW
BLK = 128
ALPHA = 0.2


def _tc1_body(nf_ref, ef_ref, wn_ref, a3_ref, wep_ref, hv_ref, scal_ref):
    hv = jnp.dot(nf_ref[...], wn_ref[...], preferred_element_type=jnp.float32)
    hv_ref[...] = hv
    scal_ref[...] = (
        jnp.dot(hv, a3_ref[...], preferred_element_type=jnp.float32)
        + jnp.dot(ef_ref[...], wep_ref[...], preferred_element_type=jnp.float32))


def _tc2_body(eft_ref, wt_ref, at_ref, q2_ref):
    w = wt_ref[:, 0:16]
    a = at_ref[0:1, 0:16]
    rows = []
    for b in range(8):
        t = jnp.dot(w, eft_ref[b], preferred_element_type=jnp.float32)
        rows.append(jnp.dot(a, t, preferred_element_type=jnp.float32))
    q2_ref[...] = jnp.concatenate(rows, axis=0)


def _tc3_body(pe_ref, nn_ref, ne_ref, esum_ref, vn_ref, ve_ref):
    esum_ref[...] = pe_ref[0] + pe_ref[1]
    denom = jnp.float32(TWOE)
    x = nn_ref[...]
    m = jnp.sum(x) / denom
    vn_ref[...] = (jnp.sum((x - m) * (x - m)) / (denom - 1.0)).reshape(1, 1)
    y = ne_ref[...]
    my = jnp.sum(y) / denom
    ve_ref[...] = (jnp.sum((y - my) * (y - my)) / (denom - 1.0)).reshape(1, 1)


def _sca_body(src_hbm, dst_hbm, q2_hbm, p1_hbm, p2_hbm, q1_hbm,
              na_out, ea_out, partial_out,
              srcv, dstv, q2v, p1v, p2v, q1v, naev, eaev,
              psumv, esumv, cntv, slab):
    c = lax.axis_index("c")
    s = lax.axis_index("s")
    wid = s * 2 + c
    base = wid * C

    pltpu.sync_copy(src_hbm.at[pl.ds(base, C)], srcv)
    pltpu.sync_copy(dst_hbm.at[pl.ds(base, C)], dstv)
    pltpu.sync_copy(q2_hbm.at[pl.ds(base, C)], q2v)
    pltpu.sync_copy(p1_hbm, p1v)
    pltpu.sync_copy(p2_hbm, p2v)
    pltpu.sync_copy(q1_hbm, q1v)

    zero16 = jnp.zeros((16,), jnp.float32)

    def zbody(i, _):
        psumv[pl.ds(i * 16, 16)] = zero16
        esumv[pl.ds(i * 16, 16)] = zero16
        cntv[pl.ds(i * 16, 16)] = zero16
        return 0
    lax.fori_loop(0, NPAD // 16, zbody, 0)

    ones16 = jnp.full((16,), 1.0, jnp.float32)

    def gbody(g, _):
        off = g * 16
        s16 = srcv[pl.ds(off, 16)]
        d16 = dstv[pl.ds(off, 16)]
        q2g = q2v[pl.ds(off, 16)]
        a = plsc.load_gather(p1v, [s16])
        b = plsc.load_gather(p2v, [d16])
        cq = plsc.load_gather(q1v, [s16])
        na = a + b
        ea = cq + q2g
        na = jnp.where(na >= 0.0, na, ALPHA * na)
        ea = jnp.where(ea >= 0.0, ea, ALPHA * ea)
        na = jnp.minimum(jnp.maximum(na, -2.0), 2.0)
        ea = jnp.minimum(jnp.maximum(ea, -2.0), 2.0)
        nae = jnp.exp(na)
        eae = jnp.exp(ea)
        naev[pl.ds(off, 16)] = nae
        eaev[pl.ds(off, 16)] = eae
        plsc.addupdate_scatter(psumv, [s16], nae)
        plsc.addupdate_scatter(esumv, [s16], eae)
        plsc.addupdate_scatter(cntv, [s16], ones16)
        return 0
    lax.fori_loop(0, C // 16, gbody, 0, unroll=2)

    pltpu.sync_copy(naev, na_out.at[pl.ds(base, C)])
    pltpu.sync_copy(eaev, ea_out.at[pl.ds(base, C)])

    segw = NPAD // 16
    seg = s * segw
    for k, arr in enumerate((psumv, esumv, cntv)):
        pltpu.sync_copy(arr, slab.at[s])
        plsc.subcore_barrier()
        pltpu.sync_copy(slab.at[0, pl.ds(seg, segw)], naev.at[pl.ds(0, segw)])
        for t in range(1, 16):
            pltpu.sync_copy(slab.at[t, pl.ds(seg, segw)],
                            eaev.at[pl.ds(0, segw)])

            def abody(i, _):
                o = i * 16
                naev[pl.ds(o, 16)] = naev[pl.ds(o, 16)] + eaev[pl.ds(o, 16)]
                return 0
            lax.fori_loop(0, segw // 16, abody, 0)
        pltpu.sync_copy(naev.at[pl.ds(0, segw)],
                        partial_out.at[c, pl.ds(k * NPAD + seg, segw)])
        plsc.subcore_barrier()


_BITS = [8192, 4096, 2048, 1024, 512, 256, 128, 64, 32, 16, 8, 4, 2, 1]


def _scb1_body(na_hbm, ea_hbm, partial_hbm,
               normn_out, norme_out,
               naev, eaev, psumv, esumv, startsv, tmpv):
    c = lax.axis_index("c")
    s = lax.axis_index("s")
    wid = s * 2 + c
    base = wid * C

    pltpu.sync_copy(na_hbm.at[pl.ds(base, C)], naev)
    pltpu.sync_copy(ea_hbm.at[pl.ds(base, C)], eaev)

    def _combine(dref, off):
        pltpu.sync_copy(partial_hbm.at[0, pl.ds(off, NPAD)], dref)
        pltpu.sync_copy(partial_hbm.at[1, pl.ds(off, NPAD)], tmpv)

        def addb(i, _):
            o = i * 16
            dref[pl.ds(o, 16)] = dref[pl.ds(o, 16)] + tmpv[pl.ds(o, 16)]
            return 0
        lax.fori_loop(0, NPAD // 16, addb, 0)

    _combine(psumv, 0)
    _combine(esumv, NPAD)
    _combine(startsv, 2 * NPAD)

    def csbody(i, carry):
        o = i * 16
        cv = startsv[pl.ds(o, 16)]
        cs = plsc.cumsum(cv)
        startsv[pl.ds(o, 16)] = cs - cv + carry
        return carry + jnp.sum(cv)
    lax.fori_loop(0, NPAD // 16, csbody, jnp.float32(0.0))

    def nbody(g, _):
        off = g * 16
        nae = naev[pl.ds(off, 16)]
        eae = eaev[pl.ds(off, 16)]
        posf = (base + off + lax.iota(jnp.int32, 16)).astype(jnp.float32)
        lo = jnp.zeros((16,), jnp.int32)
        for bit in _BITS:
            probe = lo + bit
            pidx = jnp.minimum(probe, NPAD) - 1
            sval = plsc.load_gather(startsv, [pidx])
            ok = (probe <= NPAD) & (sval <= posf)
            lo = jnp.where(ok, probe, lo)
        r = jnp.maximum(lo - 1, 0)
        divn = plsc.load_gather(psumv, [r])
        dive = plsc.load_gather(esumv, [r])
        naev[pl.ds(off, 16)] = nae / divn
        eaev[pl.ds(off, 16)] = eae / dive
        return 0
    lax.fori_loop(0, C // 16, nbody, 0, unroll=2)

    pltpu.sync_copy(naev, normn_out.at[pl.ds(base, C)])
    pltpu.sync_copy(eaev, norme_out.at[pl.ds(base, C)])


NOWN = NPAD // NW
CB = 10000
NCH = TWOE // CB


def _scb2_body(src_hbm, dst_hbm, nn_hbm, hv_hbm,
               nout_hbm,
               srcv, dstv, wv, dstm, um, wm, rowsv, accf, semg, semc):
    c = lax.axis_index("c")
    s = lax.axis_index("s")
    wid = s * 2 + c
    lo = wid * NOWN
    hi = lo + NOWN

    zero16 = jnp.zeros((16,), jnp.float32)
    zero16i = jnp.zeros((16,), jnp.int32)
    iota16 = lax.iota(jnp.int32, 16)

    def z1(i, _):
        o = i * 16
        dstm[pl.ds(o, 16)] = zero16i
        um[pl.ds(o, 16)] = zero16i
        wm[pl.ds(o, 16)] = zero16
        return 0
    lax.fori_loop(0, (CB + 16) // 16, z1, 0)

    def z2(i, _):
        accf[pl.ds(i * 16, 16)] = zero16
        return 0
    lax.fori_loop(0, (NOWN * 128) // 16, z2, 0)

    def chunk_body(ch, _):
        pltpu.async_copy(src_hbm.at[pl.ds(ch * CB, CB)], srcv, semc)
        pltpu.async_copy(dst_hbm.at[pl.ds(ch * CB, CB)], dstv, semc)
        pltpu.async_copy(nn_hbm.at[pl.ds(ch * CB, CB)], wv, semc)
        pltpu.make_async_copy(src_hbm.at[pl.ds(ch * CB, CB)], srcv, semc).wait()
        pltpu.make_async_copy(dst_hbm.at[pl.ds(ch * CB, CB)], dstv, semc).wait()
        pltpu.make_async_copy(nn_hbm.at[pl.ds(ch * CB, CB)], wv, semc).wait()

        def scan(g, ptrv):
            off = g * 16
            s16 = srcv[pl.ds(off, 16)]
            m = (s16 >= lo) & (s16 < hi)
            mi = jnp.where(m, 1, 0).astype(jnp.int32)
            pos = ptrv + plsc.cumsum(mi) - mi
            plsc.store_scatter(dstm, [pos], dstv[pl.ds(off, 16)], mask=m)
            plsc.store_scatter(um, [pos], s16 - lo, mask=m)
            plsc.store_scatter(wm, [pos], wv[pl.ds(off, 16)], mask=m)
            return ptrv + plsc.all_reduce_population_count(m)
        ptrv = lax.fori_loop(0, CB // 16, scan,
                             jnp.zeros((16,), jnp.int32), unroll=2)
        ptr = jnp.max(ptrv)

        nb = (ptr + BLK - 1) // BLK

        def blk(b, _):
            pltpu.async_copy(hv_hbm.at[dstm.at[pl.ds(b * BLK, BLK)]],
                             rowsv, semg).wait()
            rows_here = jnp.minimum(BLK, ptr - b * BLK)

            def rbody(j, _):
                e = b * BLK + j
                spl = jnp.full((16,), e, jnp.int32)
                u16 = plsc.load_gather(um, [spl])
                w16 = plsc.load_gather(wm, [spl])
                base16 = u16 * 128 + iota16
                for sg in range(8):
                    o = sg * 16
                    val = rowsv[j, pl.ds(o, 16)] * w16
                    plsc.addupdate_scatter(accf, [base16 + o], val)
                return 0
            lax.fori_loop(0, rows_here, rbody, 0)
            return 0
        lax.fori_loop(0, nb, blk, 0)
        return 0
    lax.fori_loop(0, NCH, chunk_body, 0)

    pltpu.sync_copy(accf, nout_hbm.at[pl.ds(lo * 128, NOWN * 128)])


def _scc_body(src_hbm, dst_hbm, ne_hbm, evt_hbm, partial_out,
              srcv, dstv, nev, colv, colv2, accv, accv2, slab):
    c = lax.axis_index("c")
    s = lax.axis_index("s")
    wid = s * 2 + c
    base = wid * C

    pltpu.sync_copy(src_hbm.at[pl.ds(base, C)], srcv)
    pltpu.sync_copy(dst_hbm.at[pl.ds(base, C)], dstv)
    pltpu.sync_copy(ne_hbm.at[pl.ds(base, C)], nev)

    zero16 = jnp.zeros((16,), jnp.float32)
    segw = NPAD // 16
    seg = s * segw

    segw2 = 2 * segw
    seg2 = s * segw2
    for k in range(8):
        pltpu.sync_copy(evt_hbm.at[2 * k], colv)
        pltpu.sync_copy(evt_hbm.at[2 * k + 1], colv2)

        def zb(i, _):
            o = i * 16
            accv[pl.ds(o, 16)] = zero16
            accv2[pl.ds(o, 16)] = zero16
            return 0
        lax.fori_loop(0, NPAD // 16, zb, 0)

        def gb(g, _):
            off = g * 16
            d16 = dstv[pl.ds(off, 16)]
            s16 = srcv[pl.ds(off, 16)]
            w = nev[pl.ds(off, 16)]
            v = plsc.load_gather(colv, [d16]) * w
            plsc.addupdate_scatter(accv, [s16], v)
            v2 = plsc.load_gather(colv2, [d16]) * w
            plsc.addupdate_scatter(accv2, [s16], v2)
            return 0
        lax.fori_loop(0, C // 16, gb, 0, unroll=2)

        pltpu.sync_copy(accv, slab.at[s, pl.ds(0, NPAD)])
        pltpu.sync_copy(accv2, slab.at[s, pl.ds(NPAD, NPAD)])
        plsc.subcore_barrier()
        pltpu.sync_copy(slab.at[0, pl.ds(seg2, segw2)],
                        colv.at[pl.ds(0, segw2)])
        for t in range(1, 16):
            pltpu.sync_copy(slab.at[t, pl.ds(seg2, segw2)],
                            accv.at[pl.ds(0, segw2)])

            def ab(i, _):
                o = i * 16
                colv[pl.ds(o, 16)] = colv[pl.ds(o, 16)] + accv[pl.ds(o, 16)]
                return 0
            lax.fori_loop(0, segw2 // 16, ab, 0)
        pltpu.sync_copy(colv.at[pl.ds(0, segw2)],
                        partial_out.at[c, pl.ds(k * 2 * NPAD + seg2, segw2)])
        plsc.subcore_barrier()


@jax.jit
def kernel(node_fts, edge_fts, edges, W_node, W_edge, a_node, a_edge):
    node_fts = jnp.squeeze(node_fts)
    edge_fts = jnp.squeeze(edge_fts)
    edges = jnp.squeeze(edges)

    f32 = jnp.float32
    edges2 = edges.reshape(E, 2)
    src_d = jnp.concatenate([edges2[:, 0], edges2[:, 1]])
    dst_d = jnp.concatenate([edges2[:, 1], edges2[:, 0]])

    a3 = jnp.zeros((128, 128), f32)
    a3 = a3.at[:, 0].set(a_node[:128, 0])
    a3 = a3.at[:, 1].set(a_node[128:, 0])
    a3 = a3.at[:, 2].set(a_edge[:128, 0])
    wep = jnp.zeros((16, 128), f32).at[:, 16:32].set(W_edge)
    wT = jnp.zeros((16, 128), f32).at[:, 0:16].set(W_edge.T)
    a2eT = jnp.zeros((8, 128), f32).at[0:1, 0:16].set(a_edge[128:].T)

    hv, scal = pl.pallas_call(
        _tc1_body,
        grid=(5,),
        in_specs=[
            pl.BlockSpec((2000, 128), lambda i: (i, 0)),
            pl.BlockSpec((2000, 16), lambda i: (i, 0)),
            pl.BlockSpec((128, 128), lambda i: (0, 0)),
            pl.BlockSpec((128, 128), lambda i: (0, 0)),
            pl.BlockSpec((16, 128), lambda i: (0, 0)),
        ],
        out_specs=[
            pl.BlockSpec((2000, 128), lambda i: (i, 0)),
            pl.BlockSpec((2000, 128), lambda i: (i, 0)),
        ],
        out_shape=[
            jax.ShapeDtypeStruct((N, 128), f32),
            jax.ShapeDtypeStruct((N, 128), f32),
        ],
    )(node_fts, edge_fts[:N], W_node, a3, wep)

    eft = edge_fts.reshape(1250, 128, 16).transpose(0, 2, 1)
    eft = jnp.pad(eft, ((0, 30), (0, 0), (0, 0)))
    q2_2d = pl.pallas_call(
        _tc2_body,
        grid=(160,),
        in_specs=[
            pl.BlockSpec((8, 16, 128), lambda i: (i, 0, 0)),
            pl.BlockSpec((16, 128), lambda i: (0, 0)),
            pl.BlockSpec((8, 128), lambda i: (0, 0)),
        ],
        out_specs=pl.BlockSpec((8, 128), lambda i: (i, 0)),
        out_shape=jax.ShapeDtypeStruct((1280, 128), f32),
    )(eft, wT, a2eT)
    q2 = q2_2d.reshape(1280 * 128)[:E]
    q2_d = jnp.concatenate([q2, q2])

    p1 = jnp.pad(scal[:, 0], (0, NPAD - N))
    p2 = jnp.pad(scal[:, 1], (0, NPAD - N))
    q1 = jnp.pad(scal[:, 2], (0, NPAD - N))
    evT = jnp.pad(scal[:, 16:32], ((0, NPAD - N), (0, 0))).T

    src_d, dst_d, q2_d, p1, p2, q1, evT, hv = (
        lax.optimization_barrier(
            (src_d, dst_d, q2_d, p1, p2, q1, evT, hv)))

    mesh = plsc.VectorSubcoreMesh(core_axis_name="c", subcore_axis_name="s")
    sc_params = pltpu.CompilerParams(needs_layout_passes=False)

    sca = functools.partial(
        pl.kernel, _sca_body, mesh=mesh,
        compiler_params=sc_params,
        out_type=[
            jax.ShapeDtypeStruct((TWOE,), f32),
            jax.ShapeDtypeStruct((TWOE,), f32),
            jax.ShapeDtypeStruct((2, 3 * NPAD), f32),
        ],
        scratch_types=[
            pltpu.VMEM((C,), jnp.int32),
            pltpu.VMEM((C,), jnp.int32),
            pltpu.VMEM((C,), f32),
            pltpu.VMEM((NPAD,), f32),
            pltpu.VMEM((NPAD,), f32),
            pltpu.VMEM((NPAD,), f32),
            pltpu.VMEM((C,), f32),
            pltpu.VMEM((C,), f32),
            pltpu.VMEM((NPAD,), f32),
            pltpu.VMEM((NPAD,), f32),
            pltpu.VMEM((NPAD,), f32),
            pltpu.VMEM_SHARED((16, NPAD), f32),
        ],
    )()
    na_e, ea_e, partials = sca(src_d, dst_d, q2_d, p1, p2, q1)
    na_e, ea_e, partials = lax.optimization_barrier((na_e, ea_e, partials))

    scb1 = functools.partial(
        pl.kernel, _scb1_body, mesh=mesh,
        compiler_params=sc_params,
        out_type=[
            jax.ShapeDtypeStruct((TWOE,), f32),
            jax.ShapeDtypeStruct((TWOE,), f32),
        ],
        scratch_types=[
            pltpu.VMEM((C,), f32),
            pltpu.VMEM((C,), f32),
            pltpu.VMEM((NPAD,), f32),
            pltpu.VMEM((NPAD,), f32),
            pltpu.VMEM((NPAD,), f32),
            pltpu.VMEM((NPAD,), f32),
        ],
    )()
    normn, norme = scb1(na_e, ea_e, partials)
    normn, norme = lax.optimization_barrier((normn, norme))

    scb2 = functools.partial(
        pl.kernel, _scb2_body, mesh=mesh,
        compiler_params=sc_params,
        out_type=jax.ShapeDtypeStruct((NPAD * 128,), f32),
        scratch_types=[
            pltpu.VMEM((CB,), jnp.int32),
            pltpu.VMEM((CB,), jnp.int32),
            pltpu.VMEM((CB,), f32),
            pltpu.VMEM((CB + 16,), jnp.int32),
            pltpu.VMEM((CB + 16,), jnp.int32),
            pltpu.VMEM((CB + 16,), f32),
            pltpu.VMEM((BLK, 128), f32),
            pltpu.VMEM((NOWN * 128,), f32),
            pltpu.SemaphoreType.DMA,
            pltpu.SemaphoreType.DMA,
        ],
    )()
    nacc = scb2(src_d, dst_d, normn, hv)
    norme_b = norme

    scc = functools.partial(
        pl.kernel, _scc_body, mesh=mesh,
        compiler_params=sc_params,
        out_type=jax.ShapeDtypeStruct((2, 16 * NPAD), f32),
        scratch_types=[
            pltpu.VMEM((C,), jnp.int32),
            pltpu.VMEM((C,), jnp.int32),
            pltpu.VMEM((C,), f32),
            pltpu.VMEM((NPAD,), f32),
            pltpu.VMEM((NPAD,), f32),
            pltpu.VMEM((NPAD,), f32),
            pltpu.VMEM((NPAD,), f32),
            pltpu.VMEM_SHARED((16, 2 * NPAD), f32),
        ],
    )()
    pe = scc(src_d, dst_d, norme_b, evT).reshape(2, 16, NPAD)

    nn2 = normn.reshape(2500, 128)
    ne2 = norme.reshape(2500, 128)
    esum, varn, vare = pl.pallas_call(
        _tc3_body,
        grid=(1,),
        in_specs=[
            pl.BlockSpec((2, 16, NPAD), lambda i: (0, 0, 0)),
            pl.BlockSpec((2500, 128), lambda i: (0, 0)),
            pl.BlockSpec((2500, 128), lambda i: (0, 0)),
        ],
        out_specs=[
            pl.BlockSpec((16, NPAD), lambda i: (0, 0)),
            pl.BlockSpec((1, 1), lambda i: (0, 0)),
            pl.BlockSpec((1, 1), lambda i: (0, 0)),
        ],
        out_shape=[
            jax.ShapeDtypeStruct((16, NPAD), f32),
            jax.ShapeDtypeStruct((1, 1), f32),
            jax.ShapeDtypeStruct((1, 1), f32),
        ],
    )(pe, nn2, ne2)

    node_out = nacc.reshape(NPAD, 128)[:N]
    edge_out = esum.T[:N]
    return node_out, edge_out, varn[0, 0], vare[0, 0]


# speedup vs baseline: 1.5224x; 1.0663x over previous
import functools

import jax
import jax.numpy as jnp
from jax import lax
from jax.experimental import pallas as pl
from jax.experimental.pallas import tpu as pltpu
from jax.experimental.pallas import tpu_sc as plsc

N = 10000
NPAD = 10240
E = 160000
TWOE = 2 * E
NW = 32
C = TWOE // NW
BLK = 128
ALPHA = 0.2


def _tc1_body(nf_ref, ef_ref, wn_ref, a3_ref, wep_ref, hv_ref, scal_ref):
    hv = jnp.dot(nf_ref[...], wn_ref[...], preferred_element_type=jnp.float32)
    hv_ref[...] = hv
    scal_ref[...] = (
        jnp.dot(hv, a3_ref[...], preferred_element_type=jnp.float32)
        + jnp.dot(ef_ref[...], wep_ref[...], preferred_element_type=jnp.float32))


def _tc2_body(eft_ref, wt_ref, at_ref, q2_ref):
    w = wt_ref[:, 0:16]
    a = at_ref[0:1, 0:16]
    rows = []
    for b in range(8):
        t = jnp.dot(w, eft_ref[b], preferred_element_type=jnp.float32)
        rows.append(jnp.dot(a, t, preferred_element_type=jnp.float32))
    q2_ref[...] = jnp.concatenate(rows, axis=0)


def _tc3_body(pe_ref, nn_ref, ne_ref, esum_ref, vn_ref, ve_ref):
    esum_ref[...] = pe_ref[0] + pe_ref[1]
    denom = jnp.float32(TWOE)
    x = nn_ref[...]
    m = jnp.sum(x) / denom
    vn_ref[...] = (jnp.sum((x - m) * (x - m)) / (denom - 1.0)).reshape(1, 1)
    y = ne_ref[...]
    my = jnp.sum(y) / denom
    ve_ref[...] = (jnp.sum((y - my) * (y - my)) / (denom - 1.0)).reshape(1, 1)


def _sca_body(src_hbm, dst_hbm, q2_hbm, p1_hbm, p2_hbm, q1_hbm,
              na_out, ea_out, partial_out,
              srcv, dstv, q2v, p1v, p2v, q1v, naev, eaev,
              psumv, esumv, cntv, slab):
    c = lax.axis_index("c")
    s = lax.axis_index("s")
    wid = s * 2 + c
    base = wid * C

    pltpu.sync_copy(src_hbm.at[pl.ds(base, C)], srcv)
    pltpu.sync_copy(dst_hbm.at[pl.ds(base, C)], dstv)
    pltpu.sync_copy(q2_hbm.at[pl.ds(base, C)], q2v)
    pltpu.sync_copy(p1_hbm, p1v)
    pltpu.sync_copy(p2_hbm, p2v)
    pltpu.sync_copy(q1_hbm, q1v)

    zero16 = jnp.zeros((16,), jnp.float32)

    def zbody(i, _):
        psumv[pl.ds(i * 16, 16)] = zero16
        esumv[pl.ds(i * 16, 16)] = zero16
        cntv[pl.ds(i * 16, 16)] = zero16
        return 0
    lax.fori_loop(0, NPAD // 16, zbody, 0)

    ones16 = jnp.full((16,), 1.0, jnp.float32)

    def gbody(g):
        off = g * 16
        s16 = srcv[pl.ds(off, 16)]
        d16 = dstv[pl.ds(off, 16)]
        q2g = q2v[pl.ds(off, 16)]
        a = plsc.load_gather(p1v, [s16])
        b = plsc.load_gather(p2v, [d16])
        cq = plsc.load_gather(q1v, [s16])
        na = a + b
        ea = cq + q2g
        na = jnp.where(na >= 0.0, na, ALPHA * na)
        ea = jnp.where(ea >= 0.0, ea, ALPHA * ea)
        na = jnp.minimum(jnp.maximum(na, -2.0), 2.0)
        ea = jnp.minimum(jnp.maximum(ea, -2.0), 2.0)
        nae = jnp.exp(na)
        eae = jnp.exp(ea)
        naev[pl.ds(off, 16)] = nae
        eaev[pl.ds(off, 16)] = eae
        plsc.addupdate_scatter(psumv, [s16], nae)
        plsc.addupdate_scatter(esumv, [s16], eae)
        plsc.addupdate_scatter(cntv, [s16], ones16)
    plsc.parallel_loop(0, C // 16, 1, unroll=2)(gbody)

    pltpu.sync_copy(naev, na_out.at[pl.ds(base, C)])
    pltpu.sync_copy(eaev, ea_out.at[pl.ds(base, C)])

    segw = NPAD // 16
    seg = s * segw
    for k, arr in enumerate((psumv, esumv, cntv)):
        pltpu.sync_copy(arr, slab.at[s])
        plsc.subcore_barrier()
        pltpu.sync_copy(slab.at[0, pl.ds(seg, segw)], naev.at[pl.ds(0, segw)])
        for t in range(1, 16):
            pltpu.sync_copy(slab.at[t, pl.ds(seg, segw)],
                            eaev.at[pl.ds(0, segw)])

            def abody(i, _):
                o = i * 16
                naev[pl.ds(o, 16)] = naev[pl.ds(o, 16)] + eaev[pl.ds(o, 16)]
                return 0
            lax.fori_loop(0, segw // 16, abody, 0)
        pltpu.sync_copy(naev.at[pl.ds(0, segw)],
                        partial_out.at[c, pl.ds(k * NPAD + seg, segw)])
        plsc.subcore_barrier()


_BITS = [8192, 4096, 2048, 1024, 512, 256, 128, 64, 32, 16, 8, 4, 2, 1]


def _scb1_body(na_hbm, ea_hbm, partial_hbm,
               normn_out, norme_out,
               naev, eaev, psumv, esumv, startsv, tmpv):
    c = lax.axis_index("c")
    s = lax.axis_index("s")
    wid = s * 2 + c
    base = wid * C

    pltpu.sync_copy(na_hbm.at[pl.ds(base, C)], naev)
    pltpu.sync_copy(ea_hbm.at[pl.ds(base, C)], eaev)

    def _combine(dref, off):
        pltpu.sync_copy(partial_hbm.at[0, pl.ds(off, NPAD)], dref)
        pltpu.sync_copy(partial_hbm.at[1, pl.ds(off, NPAD)], tmpv)

        def addb(i, _):
            o = i * 16
            dref[pl.ds(o, 16)] = dref[pl.ds(o, 16)] + tmpv[pl.ds(o, 16)]
            return 0
        lax.fori_loop(0, NPAD // 16, addb, 0)

    _combine(psumv, 0)
    _combine(esumv, NPAD)
    _combine(startsv, 2 * NPAD)

    def csbody(i, carry):
        o = i * 16
        cv = startsv[pl.ds(o, 16)]
        cs = plsc.cumsum(cv)
        startsv[pl.ds(o, 16)] = cs - cv + carry
        return carry + jnp.sum(cv)
    lax.fori_loop(0, NPAD // 16, csbody, jnp.float32(0.0))

    def nbody(g):
        off = g * 16
        nae = naev[pl.ds(off, 16)]
        eae = eaev[pl.ds(off, 16)]
        posf = (base + off + lax.iota(jnp.int32, 16)).astype(jnp.float32)
        lo = jnp.zeros((16,), jnp.int32)
        for bit in _BITS:
            probe = lo + bit
            pidx = jnp.minimum(probe, NPAD) - 1
            sval = plsc.load_gather(startsv, [pidx])
            ok = (probe <= NPAD) & (sval <= posf)
            lo = jnp.where(ok, probe, lo)
        r = jnp.maximum(lo - 1, 0)
        divn = plsc.load_gather(psumv, [r])
        dive = plsc.load_gather(esumv, [r])
        naev[pl.ds(off, 16)] = nae / divn
        eaev[pl.ds(off, 16)] = eae / dive
    plsc.parallel_loop(0, C // 16, 1, unroll=2)(nbody)

    pltpu.sync_copy(naev, normn_out.at[pl.ds(base, C)])
    pltpu.sync_copy(eaev, norme_out.at[pl.ds(base, C)])


NOWN = NPAD // NW
CB = 10000
NCH = TWOE // CB


def _scb2_body(src_hbm, dst_hbm, nn_hbm, hv_hbm,
               nout_hbm,
               srcv, dstv, wv, dstm, um, wm, rowsv, accf, semg, semc):
    c = lax.axis_index("c")
    s = lax.axis_index("s")
    wid = s * 2 + c
    lo = wid * NOWN
    hi = lo + NOWN

    zero16 = jnp.zeros((16,), jnp.float32)
    zero16i = jnp.zeros((16,), jnp.int32)
    iota16 = lax.iota(jnp.int32, 16)

    def z1(i, _):
        o = i * 16
        dstm[pl.ds(o, 16)] = zero16i
        um[pl.ds(o, 16)] = zero16i
        wm[pl.ds(o, 16)] = zero16
        return 0
    lax.fori_loop(0, (CB + 16) // 16, z1, 0)

    def z2(i, _):
        accf[pl.ds(i * 16, 16)] = zero16
        return 0
    lax.fori_loop(0, (NOWN * 128) // 16, z2, 0)

    def chunk_body(ch, _):
        pltpu.async_copy(src_hbm.at[pl.ds(ch * CB, CB)], srcv, semc)
        pltpu.async_copy(dst_hbm.at[pl.ds(ch * CB, CB)], dstv, semc)
        pltpu.async_copy(nn_hbm.at[pl.ds(ch * CB, CB)], wv, semc)
        pltpu.make_async_copy(src_hbm.at[pl.ds(ch * CB, CB)], srcv, semc).wait()
        pltpu.make_async_copy(dst_hbm.at[pl.ds(ch * CB, CB)], dstv, semc).wait()
        pltpu.make_async_copy(nn_hbm.at[pl.ds(ch * CB, CB)], wv, semc).wait()

        def scan(g, ptrv):
            off = g * 16
            s16 = srcv[pl.ds(off, 16)]
            m = (s16 >= lo) & (s16 < hi)
            mi = jnp.where(m, 1, 0).astype(jnp.int32)
            pos = ptrv + plsc.cumsum(mi) - mi
            plsc.store_scatter(dstm, [pos], dstv[pl.ds(off, 16)], mask=m)
            plsc.store_scatter(um, [pos], s16 - lo, mask=m)
            plsc.store_scatter(wm, [pos], wv[pl.ds(off, 16)], mask=m)
            return ptrv + plsc.all_reduce_population_count(m)
        ptrv = lax.fori_loop(0, CB // 16, scan,
                             jnp.zeros((16,), jnp.int32), unroll=2)
        ptr = jnp.max(ptrv)

        nb = (ptr + BLK - 1) // BLK

        def blk(b, _):
            pltpu.async_copy(hv_hbm.at[dstm.at[pl.ds(b * BLK, BLK)]],
                             rowsv, semg).wait()
            rows_here = jnp.minimum(BLK, ptr - b * BLK)

            @plsc.parallel_loop(0, rows_here, 1, unroll=2)
            def rbody(j):
                e = b * BLK + j
                spl = jnp.full((16,), e, jnp.int32)
                u16 = plsc.load_gather(um, [spl])
                w16 = plsc.load_gather(wm, [spl])
                base16 = u16 * 128 + iota16
                for sg in range(8):
                    o = sg * 16
                    val = rowsv[j, pl.ds(o, 16)] * w16
                    plsc.addupdate_scatter(accf, [base16 + o], val)
            return 0
        lax.fori_loop(0, nb, blk, 0)
        return 0
    lax.fori_loop(0, NCH, chunk_body, 0)

    pltpu.sync_copy(accf, nout_hbm.at[pl.ds(lo * 128, NOWN * 128)])


def _scc_body(src_hbm, dst_hbm, ne_hbm, evt_hbm, partial_out,
              srcv, dstv, nev, colv, colv2, accv, accv2, slab):
    c = lax.axis_index("c")
    s = lax.axis_index("s")
    wid = s * 2 + c
    base = wid * C

    pltpu.sync_copy(src_hbm.at[pl.ds(base, C)], srcv)
    pltpu.sync_copy(dst_hbm.at[pl.ds(base, C)], dstv)
    pltpu.sync_copy(ne_hbm.at[pl.ds(base, C)], nev)

    zero16 = jnp.zeros((16,), jnp.float32)
    segw = NPAD // 16
    seg = s * segw

    segw2 = 2 * segw
    seg2 = s * segw2
    for k in range(8):
        pltpu.sync_copy(evt_hbm.at[2 * k], colv)
        pltpu.sync_copy(evt_hbm.at[2 * k + 1], colv2)

        def zb(i, _):
            o = i * 16
            accv[pl.ds(o, 16)] = zero16
            accv2[pl.ds(o, 16)] = zero16
            return 0
        lax.fori_loop(0, NPAD // 16, zb, 0)

        def gb(g):
            off = g * 16
            d16 = dstv[pl.ds(off, 16)]
            s16 = srcv[pl.ds(off, 16)]
            w = nev[pl.ds(off, 16)]
            v = plsc.load_gather(colv, [d16]) * w
            plsc.addupdate_scatter(accv, [s16], v)
            v2 = plsc.load_gather(colv2, [d16]) * w
            plsc.addupdate_scatter(accv2, [s16], v2)
        plsc.parallel_loop(0, C // 16, 1, unroll=2)(gb)

        pltpu.sync_copy(accv, slab.at[s, pl.ds(0, NPAD)])
        pltpu.sync_copy(accv2, slab.at[s, pl.ds(NPAD, NPAD)])
        plsc.subcore_barrier()
        pltpu.sync_copy(slab.at[0, pl.ds(seg2, segw2)],
                        colv.at[pl.ds(0, segw2)])
        for t in range(1, 16):
            pltpu.sync_copy(slab.at[t, pl.ds(seg2, segw2)],
                            accv.at[pl.ds(0, segw2)])

            def ab(i, _):
                o = i * 16
                colv[pl.ds(o, 16)] = colv[pl.ds(o, 16)] + accv[pl.ds(o, 16)]
                return 0
            lax.fori_loop(0, segw2 // 16, ab, 0)
        pltpu.sync_copy(colv.at[pl.ds(0, segw2)],
                        partial_out.at[c, pl.ds(k * 2 * NPAD + seg2, segw2)])
        plsc.subcore_barrier()


@jax.jit
def kernel(node_fts, edge_fts, edges, W_node, W_edge, a_node, a_edge):
    node_fts = jnp.squeeze(node_fts)
    edge_fts = jnp.squeeze(edge_fts)
    edges = jnp.squeeze(edges)

    f32 = jnp.float32
    edges2 = edges.reshape(E, 2)
    src_d = jnp.concatenate([edges2[:, 0], edges2[:, 1]])
    dst_d = jnp.concatenate([edges2[:, 1], edges2[:, 0]])

    a3 = jnp.zeros((128, 128), f32)
    a3 = a3.at[:, 0].set(a_node[:128, 0])
    a3 = a3.at[:, 1].set(a_node[128:, 0])
    a3 = a3.at[:, 2].set(a_edge[:128, 0])
    wep = jnp.zeros((16, 128), f32).at[:, 16:32].set(W_edge)
    wT = jnp.zeros((16, 128), f32).at[:, 0:16].set(W_edge.T)
    a2eT = jnp.zeros((8, 128), f32).at[0:1, 0:16].set(a_edge[128:].T)

    hv, scal = pl.pallas_call(
        _tc1_body,
        grid=(5,),
        in_specs=[
            pl.BlockSpec((2000, 128), lambda i: (i, 0)),
            pl.BlockSpec((2000, 16), lambda i: (i, 0)),
            pl.BlockSpec((128, 128), lambda i: (0, 0)),
            pl.BlockSpec((128, 128), lambda i: (0, 0)),
            pl.BlockSpec((16, 128), lambda i: (0, 0)),
        ],
        out_specs=[
            pl.BlockSpec((2000, 128), lambda i: (i, 0)),
            pl.BlockSpec((2000, 128), lambda i: (i, 0)),
        ],
        out_shape=[
            jax.ShapeDtypeStruct((N, 128), f32),
            jax.ShapeDtypeStruct((N, 128), f32),
        ],
    )(node_fts, edge_fts[:N], W_node, a3, wep)

    eft = edge_fts.reshape(1250, 128, 16).transpose(0, 2, 1)
    eft = jnp.pad(eft, ((0, 30), (0, 0), (0, 0)))
    q2_2d = pl.pallas_call(
        _tc2_body,
        grid=(160,),
        in_specs=[
            pl.BlockSpec((8, 16, 128), lambda i: (i, 0, 0)),
            pl.BlockSpec((16, 128), lambda i: (0, 0)),
            pl.BlockSpec((8, 128), lambda i: (0, 0)),
        ],
        out_specs=pl.BlockSpec((8, 128), lambda i: (i, 0)),
        out_shape=jax.ShapeDtypeStruct((1280, 128), f32),
    )(eft, wT, a2eT)
    q2 = q2_2d.reshape(1280 * 128)[:E]
    q2_d = jnp.concatenate([q2, q2])

    p1 = jnp.pad(scal[:, 0], (0, NPAD - N))
    p2 = jnp.pad(scal[:, 1], (0, NPAD - N))
    q1 = jnp.pad(scal[:, 2], (0, NPAD - N))
    evT = jnp.pad(scal[:, 16:32], ((0, NPAD - N), (0, 0))).T

    src_d, dst_d, q2_d, p1, p2, q1, evT, hv = (
        lax.optimization_barrier(
            (src_d, dst_d, q2_d, p1, p2, q1, evT, hv)))

    mesh = plsc.VectorSubcoreMesh(core_axis_name="c", subcore_axis_name="s")
    sc_params = pltpu.CompilerParams(needs_layout_passes=False)

    sca = functools.partial(
        pl.kernel, _sca_body, mesh=mesh,
        compiler_params=sc_params,
        out_type=[
            jax.ShapeDtypeStruct((TWOE,), f32),
            jax.ShapeDtypeStruct((TWOE,), f32),
            jax.ShapeDtypeStruct((2, 3 * NPAD), f32),
        ],
        scratch_types=[
            pltpu.VMEM((C,), jnp.int32),
            pltpu.VMEM((C,), jnp.int32),
            pltpu.VMEM((C,), f32),
            pltpu.VMEM((NPAD,), f32),
            pltpu.VMEM((NPAD,), f32),
            pltpu.VMEM((NPAD,), f32),
            pltpu.VMEM((C,), f32),
            pltpu.VMEM((C,), f32),
            pltpu.VMEM((NPAD,), f32),
            pltpu.VMEM((NPAD,), f32),
            pltpu.VMEM((NPAD,), f32),
            pltpu.VMEM_SHARED((16, NPAD), f32),
        ],
    )()
    na_e, ea_e, partials = sca(src_d, dst_d, q2_d, p1, p2, q1)
    na_e, ea_e, partials = lax.optimization_barrier((na_e, ea_e, partials))

    scb1 = functools.partial(
        pl.kernel, _scb1_body, mesh=mesh,
        compiler_params=sc_params,
        out_type=[
            jax.ShapeDtypeStruct((TWOE,), f32),
            jax.ShapeDtypeStruct((TWOE,), f32),
        ],
        scratch_types=[
            pltpu.VMEM((C,), f32),
            pltpu.VMEM((C,), f32),
            pltpu.VMEM((NPAD,), f32),
            pltpu.VMEM((NPAD,), f32),
            pltpu.VMEM((NPAD,), f32),
            pltpu.VMEM((NPAD,), f32),
        ],
    )()
    normn, norme = scb1(na_e, ea_e, partials)
    normn, norme = lax.optimization_barrier((normn, norme))

    scb2 = functools.partial(
        pl.kernel, _scb2_body, mesh=mesh,
        compiler_params=sc_params,
        out_type=jax.ShapeDtypeStruct((NPAD * 128,), f32),
        scratch_types=[
            pltpu.VMEM((CB,), jnp.int32),
            pltpu.VMEM((CB,), jnp.int32),
            pltpu.VMEM((CB,), f32),
            pltpu.VMEM((CB + 16,), jnp.int32),
            pltpu.VMEM((CB + 16,), jnp.int32),
            pltpu.VMEM((CB + 16,), f32),
            pltpu.VMEM((BLK, 128), f32),
            pltpu.VMEM((NOWN * 128,), f32),
            pltpu.SemaphoreType.DMA,
            pltpu.SemaphoreType.DMA,
        ],
    )()
    nacc = scb2(src_d, dst_d, normn, hv)
    norme_b = norme

    scc = functools.partial(
        pl.kernel, _scc_body, mesh=mesh,
        compiler_params=sc_params,
        out_type=jax.ShapeDtypeStruct((2, 16 * NPAD), f32),
        scratch_types=[
            pltpu.VMEM((C,), jnp.int32),
            pltpu.VMEM((C,), jnp.int32),
            pltpu.VMEM((C,), f32),
            pltpu.VMEM((NPAD,), f32),
            pltpu.VMEM((NPAD,), f32),
            pltpu.VMEM((NPAD,), f32),
            pltpu.VMEM((NPAD,), f32),
            pltpu.VMEM_SHARED((16, 2 * NPAD), f32),
        ],
    )()
    pe = scc(src_d, dst_d, norme_b, evT).reshape(2, 16, NPAD)

    nn2 = normn.reshape(2500, 128)
    ne2 = norme.reshape(2500, 128)
    esum, varn, vare = pl.pallas_call(
        _tc3_body,
        grid=(1,),
        in_specs=[
            pl.BlockSpec((2, 16, NPAD), lambda i: (0, 0, 0)),
            pl.BlockSpec((2500, 128), lambda i: (0, 0)),
            pl.BlockSpec((2500, 128), lambda i: (0, 0)),
        ],
        out_specs=[
            pl.BlockSpec((16, NPAD), lambda i: (0, 0)),
            pl.BlockSpec((1, 1), lambda i: (0, 0)),
            pl.BlockSpec((1, 1), lambda i: (0, 0)),
        ],
        out_shape=[
            jax.ShapeDtypeStruct((16, NPAD), f32),
            jax.ShapeDtypeStruct((1, 1), f32),
            jax.ShapeDtypeStruct((1, 1), f32),
        ],
    )(pe, nn2, ne2)

    node_out = nacc.reshape(NPAD, 128)[:N]
    edge_out = esum.T[:N]
    return node_out, edge_out, varn[0, 0], vare[0, 0]
